# Initial kernel scaffold; baseline (speedup 1.0000x reference)
import jax
import jax.numpy as jnp
from jax.experimental import pallas as pl


def kernel(input_ids, frozen_table, trainable_table):
    raise NotImplementedError("write your pallas kernel here")


# speedup vs baseline: 7.5356x; 7.5356x over previous
import jax
import jax.numpy as jnp
from jax import lax
from jax.experimental import pallas as pl
from jax.experimental.pallas import tpu as pltpu
from jax.experimental.pallas import tpu_sc as plsc

_NC = 2
_NS = 16
_NW = _NC * _NS
_SUB = 128
_CHUNK = 4


def _sc_gather(table, ids2d):
    nb, sub = ids2d.shape
    d = table.shape[1]
    assert sub == _SUB and nb % (_NW * _CHUNK) == 0
    bpw = nb // _NW
    steps = bpw // _CHUNK

    mesh = plsc.VectorSubcoreMesh(core_axis_name="c", subcore_axis_name="s",
                                  num_cores=_NC, num_subcores=_NS)

    @pl.kernel(
        out_type=jax.ShapeDtypeStruct((nb, _SUB, d), jnp.float32),
        mesh=mesh,
        scratch_types=[
            pltpu.VMEM((bpw, _SUB), jnp.int32),
            pltpu.VMEM((_CHUNK, _SUB, d), jnp.float32),
            pltpu.SemaphoreType.DMA,
        ],
        compiler_params=pltpu.CompilerParams(use_tc_tiling_on_sc=False),
    )
    def k(table_hbm, ids_hbm, out_hbm, idx_v, rows_v, gsem):
        wid = lax.axis_index("s") * _NC + lax.axis_index("c")
        blk0 = wid * bpw
        pltpu.sync_copy(ids_hbm.at[pl.ds(blk0, bpw)], idx_v)

        def step(g, carry):
            b = g * _CHUNK
            cps = [
                pltpu.async_copy(table_hbm.at[idx_v.at[b + j]], rows_v.at[j], gsem)
                for j in range(_CHUNK)
            ]
            for cp in cps:
                cp.wait()
            pltpu.sync_copy(rows_v, out_hbm.at[pl.ds(blk0 + b, _CHUNK)])
            return carry

        lax.fori_loop(0, steps, step, 0)

    return k(table, ids2d)


def kernel(input_ids, frozen_table, trainable_table):
    shape = input_ids.shape
    d = frozen_table.shape[-1]
    ids = input_ids.reshape(-1).astype(jnp.int32)
    table = jnp.concatenate([frozen_table, trainable_table], axis=0)
    out = _sc_gather(table, ids.reshape(-1, _SUB))
    return out.reshape(shape + (d,))


# speedup vs baseline: 7.8289x; 1.0389x over previous
import jax
import jax.numpy as jnp
from jax import lax
from jax.experimental import pallas as pl
from jax.experimental.pallas import tpu as pltpu
from jax.experimental.pallas import tpu_sc as plsc

_NC = 2
_NS = 16
_NW = _NC * _NS
_SUB = 128
_CHUNK = 4


def _sc_gather(table, ids2d):
    nb, sub = ids2d.shape
    d = table.shape[1]
    assert sub == _SUB and nb % (_NW * _CHUNK) == 0
    bpw = nb // _NW
    steps = bpw // _CHUNK

    mesh = plsc.VectorSubcoreMesh(core_axis_name="c", subcore_axis_name="s",
                                  num_cores=_NC, num_subcores=_NS)

    @pl.kernel(
        out_type=jax.ShapeDtypeStruct((nb, _SUB, d), jnp.float32),
        mesh=mesh,
        scratch_types=[
            pltpu.VMEM((bpw, _SUB), jnp.int32),
            pltpu.VMEM((2, _CHUNK, _SUB, d), jnp.float32),
            pltpu.SemaphoreType.DMA((2,)),
            pltpu.SemaphoreType.DMA((2,)),
        ],
        compiler_params=pltpu.CompilerParams(use_tc_tiling_on_sc=False),
    )
    def k(table_hbm, ids_hbm, out_hbm, idx_v, rows_v, gsem, osem):
        wid = lax.axis_index("s") * _NC + lax.axis_index("c")
        blk0 = wid * bpw
        pltpu.sync_copy(ids_hbm.at[pl.ds(blk0, bpw)], idx_v)

        def fire_gathers(g, buf):
            b = g * _CHUNK
            for j in range(_CHUNK):
                pltpu.async_copy(
                    table_hbm.at[idx_v.at[b + j]], rows_v.at[buf, j], gsem.at[buf]
                )

        def drain_gathers(buf):
            for j in range(_CHUNK):
                pltpu.make_async_copy(
                    table_hbm.at[idx_v.at[j]], rows_v.at[buf, j], gsem.at[buf]
                ).wait()

        def fire_out(g, buf):
            pltpu.async_copy(
                rows_v.at[buf], out_hbm.at[pl.ds(blk0 + g * _CHUNK, _CHUNK)],
                osem.at[buf],
            )

        def drain_out(g, buf):
            pltpu.make_async_copy(
                rows_v.at[buf], out_hbm.at[pl.ds(blk0 + g * _CHUNK, _CHUNK)],
                osem.at[buf],
            ).wait()

        fire_gathers(0, 0)

        def step(g, carry):
            buf = lax.rem(g, 2)
            nbuf = 1 - buf
            drain_gathers(buf)

            @pl.when(g >= 1)
            def _():
                drain_out(g - 1, nbuf)

            @pl.when(g + 1 < steps)
            def _():
                fire_gathers(g + 1, nbuf)

            fire_out(g, buf)
            return carry

        lax.fori_loop(0, steps, step, 0)
        drain_out(steps - 1, lax.rem(steps - 1, 2))

    return k(table, ids2d)


def kernel(input_ids, frozen_table, trainable_table):
    shape = input_ids.shape
    d = frozen_table.shape[-1]
    ids = input_ids.reshape(-1).astype(jnp.int32)
    table = jnp.concatenate([frozen_table, trainable_table], axis=0)
    out = _sc_gather(table, ids.reshape(-1, _SUB))
    return out.reshape(shape + (d,))
